# baseline (speedup 1.0000x reference)
import functools

import jax
import jax.numpy as jnp
from jax import lax
from jax.experimental import pallas as pl
from jax.experimental.pallas import tpu as pltpu
from jax.experimental.pallas import tpu_sc as plsc

N_ITEM = 1000000
N_DIM = 64
BATCH = 16384

NC = 2
NS = 16
L = 16
NW = NC * NS
B_PER_W = BATCH // NW
CHUNK = 128
N_CHUNKS = B_PER_W // CHUNK
PAIR_W = 2 * N_DIM
SPLIT = 524288
LO_PAIRS = SPLIT // 2
HI_PAIRS = (N_ITEM - SPLIT) // 2


def _sc_body(word_hbm, ctx_hbm, wlo_hbm, whi_hbm, clo_hbm, chi_hbm, out_hbm,
             widx, cidx, wplo, wphi, cplo, cphi,
             wblo, wbhi, cblo, cbhi, out_v, sems):
    wid = lax.axis_index("s") * NC + lax.axis_index("c")
    base = wid * B_PER_W

    pltpu.sync_copy(word_hbm.at[pl.ds(base, B_PER_W)], widx)
    pltpu.sync_copy(ctx_hbm.at[pl.ds(base, B_PER_W)], cidx)

    for k in range(B_PER_W // L):
        sl = pl.ds(k * L, L)
        wp = widx[sl] >> 1
        cp = cidx[sl] >> 1
        wplo[sl] = jnp.minimum(wp, LO_PAIRS - 1)
        wphi[sl] = jnp.clip(wp - LO_PAIRS, 0, HI_PAIRS - 1)
        cplo[sl] = jnp.minimum(cp, LO_PAIRS - 1)
        cphi[sl] = jnp.clip(cp - LO_PAIRS, 0, HI_PAIRS - 1)

    for c in range(N_CHUNKS):
        sl = pl.ds(c * CHUNK, CHUNK)
        copies = [
            pltpu.async_copy(wlo_hbm.at[wplo.at[sl]], wblo, sems.at[0]),
            pltpu.async_copy(whi_hbm.at[wphi.at[sl]], wbhi, sems.at[1]),
            pltpu.async_copy(clo_hbm.at[cplo.at[sl]], cblo, sems.at[2]),
            pltpu.async_copy(chi_hbm.at[cphi.at[sl]], cbhi, sems.at[3]),
        ]
        for cp_ in copies:
            cp_.wait()

        def body(g, _, c=c):
            gbase = c * CHUNK + g * L
            ridx = jnp.arange(L, dtype=jnp.int32) + g * L
            wi = widx[pl.ds(gbase, L)]
            ci = cidx[pl.ds(gbase, L)]
            wcol = (wi & 1) * N_DIM
            ccol = (ci & 1) * N_DIM
            wlo_m = wi < SPLIT
            clo_m = ci < SPLIT
            acc = jnp.zeros((L,), jnp.float32)
            for j in range(N_DIM):
                w_l = plsc.load_gather(wblo, [ridx, wcol + j])
                w_h = plsc.load_gather(wbhi, [ridx, wcol + j])
                x_l = plsc.load_gather(cblo, [ridx, ccol + j])
                x_h = plsc.load_gather(cbhi, [ridx, ccol + j])
                w = jnp.where(wlo_m, w_l, w_h)
                x = jnp.where(clo_m, x_l, x_h)
                acc = acc + w * x
            out_v[pl.ds(gbase, L)] = 1.0 / (1.0 + jnp.exp(-acc))
            return 0

        lax.fori_loop(0, CHUNK // L, body, 0)

    pltpu.sync_copy(out_v, out_hbm.at[pl.ds(base, B_PER_W)])


@jax.jit
def _skipgram_sc(word, ctx, wlo, whi, clo, chi):
    mesh = plsc.VectorSubcoreMesh(core_axis_name="c", subcore_axis_name="s",
                                  num_cores=NC, num_subcores=NS)
    return pl.kernel(
        _sc_body,
        out_type=jax.ShapeDtypeStruct((BATCH,), jnp.float32),
        mesh=mesh,
        compiler_params=pltpu.CompilerParams(needs_layout_passes=False),
        scratch_types=[
            pltpu.VMEM((B_PER_W,), jnp.int32),
            pltpu.VMEM((B_PER_W,), jnp.int32),
            pltpu.VMEM((B_PER_W,), jnp.int32),
            pltpu.VMEM((B_PER_W,), jnp.int32),
            pltpu.VMEM((B_PER_W,), jnp.int32),
            pltpu.VMEM((B_PER_W,), jnp.int32),
            pltpu.VMEM((CHUNK, PAIR_W), jnp.float32),
            pltpu.VMEM((CHUNK, PAIR_W), jnp.float32),
            pltpu.VMEM((CHUNK, PAIR_W), jnp.float32),
            pltpu.VMEM((CHUNK, PAIR_W), jnp.float32),
            pltpu.VMEM((B_PER_W,), jnp.float32),
            pltpu.SemaphoreType.DMA((4,)),
        ],
    )(word, ctx, wlo, whi, clo, chi)


def kernel(word, context, word_embeddings, context_embeddings):
    wlo = word_embeddings[:SPLIT].reshape(LO_PAIRS, PAIR_W)
    whi = word_embeddings[SPLIT:].reshape(HI_PAIRS, PAIR_W)
    clo = context_embeddings[:SPLIT].reshape(LO_PAIRS, PAIR_W)
    chi = context_embeddings[SPLIT:].reshape(HI_PAIRS, PAIR_W)
    return _skipgram_sc(word.astype(jnp.int32), context.astype(jnp.int32),
                        wlo, whi, clo, chi)
